# baseline (speedup 1.0000x reference)
import functools

import jax
import jax.numpy as jnp
import numpy as np
from jax import lax
from jax.experimental import pallas as pl
from jax.experimental.pallas import tpu as pltpu
from jax.experimental.pallas import tpu_sc as plsc

N = 10000
E = 320000
D = 128

NC = 2
NS = 16
NW = NC * NS
E_PER_W = E // NW
B = 64
FULL_STEPS = 156
BT = E_PER_W - FULL_STEPS * B
NP = 10240
R_PER_T = NP // NS

_PERM = np.empty((D,), dtype=np.int32)
for _j in range(D // 2):
    _gq, _k = _j // 16, _j % 16
    _PERM[_j] = _gq * 32 + _k
    _PERM[D // 2 + _j] = _gq * 32 + 16 + _k


def _msg_body(x_ref, wt_ref, m_ref):
    e = jnp.dot(x_ref[...], wt_ref[...], preferred_element_type=jnp.float32)
    g = jax.nn.sigmoid(e[:, :D])
    m = g * e[:, D:]
    u = lax.bitcast_convert_type(m, jnp.uint32)
    rb = (u + jnp.uint32(0x7FFF) + ((u >> 16) & jnp.uint32(1))) >> 16
    packed = rb[:, : D // 2] | (rb[:, D // 2:] << 16)
    m_ref[...] = lax.bitcast_convert_type(packed, jnp.int32)


def _msg(x, wt):
    blk = 1000
    return pl.pallas_call(
        _msg_body,
        grid=(N // blk,),
        in_specs=[
            pl.BlockSpec((blk, D), lambda i: (i, 0)),
            pl.BlockSpec((D, 2 * D), lambda i: (0, 0)),
        ],
        out_specs=pl.BlockSpec((blk, D // 2), lambda i: (i, 0)),
        out_shape=jax.ShapeDtypeStruct((N, D // 2), jnp.int32),
    )(x, wt)


def _make_scatter():
    mesh = plsc.VectorSubcoreMesh(core_axis_name="c", subcore_axis_name="s")

    @functools.partial(
        pl.kernel,
        out_type=jax.ShapeDtypeStruct((NC, NP, D), jnp.float32),
        mesh=mesh,
        compiler_params=pltpu.CompilerParams(
            needs_layout_passes=False, use_tc_tiling_on_sc=False),
        scratch_types=[
            pltpu.VMEM((E_PER_W,), jnp.int32),
            pltpu.VMEM((E_PER_W,), jnp.int32),
            pltpu.VMEM((B, D // 2), jnp.int32),
            pltpu.VMEM((B, D // 2), jnp.int32),
            pltpu.VMEM((B, D), jnp.float32),
            pltpu.VMEM((B, D), jnp.float32),
            pltpu.VMEM((BT, D // 2), jnp.int32),
            pltpu.VMEM((BT, D), jnp.float32),
            pltpu.VMEM_SHARED((NP, D), jnp.float32),
            pltpu.SemaphoreType.DMA,
            pltpu.SemaphoreType.DMA,
            pltpu.SemaphoreType.DMA,
            pltpu.SemaphoreType.DMA,
            pltpu.SemaphoreType.DMA,
        ],
    )
    def scatter_k(m_hbm, src_hbm, tgt_hbm, init_hbm, out_hbm,
                  tgt_v, src_v, pb0, pb1, fb0, fb1, pbt, fbt, acc,
                  semg0, semg1, sems0, sems1, semt):
        c = lax.axis_index("c")
        s = lax.axis_index("s")
        wid = s * NC + c

        cp_t = pltpu.async_copy(tgt_hbm.at[wid], tgt_v, semg0)
        cp_s = pltpu.async_copy(src_hbm.at[wid], src_v, semg1)

        rbase = s * R_PER_T
        pltpu.sync_copy(init_hbm.at[pl.ds(rbase, R_PER_T)],
                        acc.at[pl.ds(rbase, R_PER_T)])
        cp_t.wait()
        cp_s.wait()
        plsc.subcore_barrier()

        def wait_g(buf, sem):
            pltpu.make_async_copy(m_hbm.at[pl.ds(0, B)], buf, sem).wait()

        def wait_s(buf, sem):
            pltpu.make_async_copy(buf, acc.at[pl.ds(0, B)], sem).wait()

        def tslice(ref, i, n):
            return ref.at[pl.ds(i * B, n)]

        def gather(i, buf, sem):
            pltpu.async_copy(m_hbm.at[tslice(tgt_v, i, B)], buf, sem)

        MASK = jnp.int32(-65536)

        def widen(pb, fb, nrows):
            def row8(r8, carry):
                for dr in range(8):
                    r = r8 * 8 + dr
                    for gq in range(D // 32):
                        v = pb[r, pl.ds(gq * 16, 16)]
                        lo = plsc.bitcast(lax.shift_left(v, 16), jnp.float32)
                        hi = plsc.bitcast(lax.bitwise_and(v, MASK), jnp.float32)
                        fb[r, pl.ds(gq * 32, 16)] = lo
                        fb[r, pl.ds(gq * 32 + 16, 16)] = hi
                return carry

            lax.fori_loop(0, nrows // 8, row8, 0)

        gather(0, pb0, semg0)
        gather(1, pb1, semg1)
        pltpu.async_copy(
            m_hbm.at[tgt_v.at[pl.ds(FULL_STEPS * B, BT)]], pbt, semt)

        wait_g(pb0, semg0)
        widen(pb0, fb0, B)
        pltpu.async_copy(fb0, acc.at[tslice(src_v, 0, B)], sems0, add=True)
        gather(2, pb0, semg0)

        wait_g(pb1, semg1)
        widen(pb1, fb1, B)
        pltpu.async_copy(fb1, acc.at[tslice(src_v, 1, B)], sems1, add=True)
        gather(3, pb1, semg1)

        def pair(kk, carry):
            i = 2 + 2 * kk

            wait_g(pb0, semg0)
            wait_s(fb0, sems0)
            widen(pb0, fb0, B)
            pltpu.async_copy(fb0, acc.at[tslice(src_v, i, B)], sems0, add=True)

            @pl.when(i + 2 < FULL_STEPS)
            def _():
                gather(i + 2, pb0, semg0)

            wait_g(pb1, semg1)
            wait_s(fb1, sems1)
            widen(pb1, fb1, B)
            pltpu.async_copy(fb1, acc.at[tslice(src_v, i + 1, B)], sems1,
                             add=True)

            @pl.when(i + 3 < FULL_STEPS)
            def _():
                gather(i + 3, pb1, semg1)

            return carry

        lax.fori_loop(0, (FULL_STEPS - 2) // 2, pair, 0)

        pltpu.make_async_copy(m_hbm.at[pl.ds(0, BT)], pbt, semt).wait()
        widen(pbt, fbt, BT)
        wait_s(fb0, sems0)
        wait_s(fb1, sems1)
        pltpu.sync_copy(
            fbt, acc.at[src_v.at[pl.ds(FULL_STEPS * B, BT)]], add=True)
        plsc.subcore_barrier()

        pltpu.sync_copy(acc.at[pl.ds(rbase, R_PER_T)],
                        out_hbm.at[c, pl.ds(rbase, R_PER_T)])

    return scatter_k


_scatter_k = _make_scatter()


def _add_body(x_ref, p_ref, o_ref):
    o_ref[...] = x_ref[...] + p_ref[0] + p_ref[1]


def _combine(x, p):
    blk = 1000
    return pl.pallas_call(
        _add_body,
        grid=(N // blk,),
        in_specs=[
            pl.BlockSpec((blk, D), lambda i: (i, 0)),
            pl.BlockSpec((NC, blk, D), lambda i: (0, i, 0)),
        ],
        out_specs=pl.BlockSpec((blk, D), lambda i: (i, 0)),
        out_shape=jax.ShapeDtypeStruct((N, D), jnp.float32),
    )(x, p)


def kernel(input, edge_sources, edge_targets, distance_nbr, W):
    x = input
    perm = jnp.asarray(_PERM)
    wp = jnp.concatenate([W[:D][perm], W[D:][perm]], axis=0)
    m = _msg(x, wp.T)
    src = edge_sources.astype(jnp.int32).reshape(NW, E_PER_W)
    tgt = edge_targets.astype(jnp.int32).reshape(NW, E_PER_W)
    init = jnp.zeros((NP, D), jnp.float32)
    p = _scatter_k(m, src, tgt, init)
    return _combine(x, p)


# speedup vs baseline: 1.7419x; 1.7419x over previous
import functools

import jax
import jax.numpy as jnp
from jax import lax
from jax.experimental import pallas as pl
from jax.experimental.pallas import tpu as pltpu
from jax.experimental.pallas import tpu_sc as plsc

N = 10000
E = 320000
D = 128

NC = 2
NS = 16
NW = NC * NS
E_PER_W = E // NW
B = 72
FULL_STEPS = 138
BT = E_PER_W - FULL_STEPS * B
NP = 10240
R_PER_T = NP // NS
RB = 80


def _msg_body(x_ref, wt_ref, m_ref):
    e = jnp.dot(x_ref[...], wt_ref[...], preferred_element_type=jnp.float32)
    g = jax.nn.sigmoid(e[:, :D])
    m_ref[...] = g * e[:, D:]


def _msg(x, wt):
    blk = 1000
    return pl.pallas_call(
        _msg_body,
        grid=(N // blk,),
        in_specs=[
            pl.BlockSpec((blk, D), lambda i: (i, 0)),
            pl.BlockSpec((D, 2 * D), lambda i: (0, 0)),
        ],
        out_specs=pl.BlockSpec((blk, D), lambda i: (i, 0)),
        out_shape=jax.ShapeDtypeStruct((N, D), jnp.float32),
    )(x, wt)


def _make_scatter():
    mesh = plsc.VectorSubcoreMesh(core_axis_name="c", subcore_axis_name="s")

    @functools.partial(
        pl.kernel,
        out_type=jax.ShapeDtypeStruct((NC, NP, D), jnp.float32),
        mesh=mesh,
        scratch_types=[
            pltpu.VMEM((E_PER_W,), jnp.int32),
            pltpu.VMEM((E_PER_W,), jnp.int32),
            pltpu.VMEM((B, D), jnp.float32),
            pltpu.VMEM((B, D), jnp.float32),
            pltpu.VMEM((B, D), jnp.float32),
            pltpu.VMEM_SHARED((NP, D), jnp.float32),
            pltpu.SemaphoreType.DMA,
            pltpu.SemaphoreType.DMA,
            pltpu.SemaphoreType.DMA,
        ],
    )
    def scatter_k(m_hbm, src_hbm, tgt_hbm, init_hbm, out_hbm,
                  tgt_v, src_v, rows0, rows1, rows2, acc,
                  semg0, semg1, semg2):
        c = lax.axis_index("c")
        s = lax.axis_index("s")
        wid = s * NC + c

        cp_t = pltpu.async_copy(tgt_hbm.at[wid], tgt_v, semg0)
        cp_s = pltpu.async_copy(src_hbm.at[wid], src_v, semg1)

        rbase = s * R_PER_T
        pltpu.sync_copy(init_hbm.at[pl.ds(rbase, R_PER_T)],
                        acc.at[pl.ds(rbase, R_PER_T)])
        cp_t.wait()
        cp_s.wait()
        plsc.subcore_barrier()

        def wait_g(buf, sem):
            pltpu.make_async_copy(m_hbm.at[pl.ds(0, B)], buf, sem).wait()

        def wait_s(buf, sem):
            pltpu.make_async_copy(buf, acc.at[pl.ds(0, B)], sem).wait()

        def tslice(ref, i):
            return ref.at[pl.ds(i * B, B)]

        pltpu.async_copy(m_hbm.at[tslice(tgt_v, 0)], rows0, semg0)
        pltpu.async_copy(m_hbm.at[tslice(tgt_v, 1)], rows1, semg1)
        pltpu.async_copy(m_hbm.at[tslice(tgt_v, 2)], rows2, semg2)

        def triple(t, carry):
            i = 3 * t
            for j, (buf, sem) in enumerate(
                    [(rows0, semg0), (rows1, semg1), (rows2, semg2)]):
                wait_g(buf, sem)
                pltpu.sync_copy(buf, acc.at[tslice(src_v, i + j)], add=True)

                @pl.when(i + j + 3 < FULL_STEPS)
                def _():
                    pltpu.async_copy(
                        m_hbm.at[tslice(tgt_v, i + j + 3)], buf, sem)

            return carry

        lax.fori_loop(0, FULL_STEPS // 3, triple, 0)
        tb = FULL_STEPS * B
        pltpu.async_copy(
            m_hbm.at[tgt_v.at[pl.ds(tb, BT)]], rows0.at[pl.ds(0, BT)], semg0)
        pltpu.make_async_copy(
            m_hbm.at[pl.ds(0, BT)], rows0.at[pl.ds(0, BT)], semg0).wait()
        pltpu.sync_copy(
            rows0.at[pl.ds(0, BT)], acc.at[src_v.at[pl.ds(tb, BT)]], add=True)
        plsc.subcore_barrier()

        pltpu.sync_copy(acc.at[pl.ds(rbase, R_PER_T)],
                        out_hbm.at[c, pl.ds(rbase, R_PER_T)])

    return scatter_k


_scatter_k = _make_scatter()


def _add_body(x_ref, p_ref, o_ref):
    o_ref[...] = x_ref[...] + p_ref[0] + p_ref[1]


def _combine(x, p):
    blk = 1000
    return pl.pallas_call(
        _add_body,
        grid=(N // blk,),
        in_specs=[
            pl.BlockSpec((blk, D), lambda i: (i, 0)),
            pl.BlockSpec((NC, blk, D), lambda i: (0, i, 0)),
        ],
        out_specs=pl.BlockSpec((blk, D), lambda i: (i, 0)),
        out_shape=jax.ShapeDtypeStruct((N, D), jnp.float32),
    )(x, p)


def kernel(input, edge_sources, edge_targets, distance_nbr, W):
    x = input
    m = _msg(x, W.T)
    src = edge_sources.astype(jnp.int32).reshape(NW, E_PER_W)
    tgt = edge_targets.astype(jnp.int32).reshape(NW, E_PER_W)
    init = jnp.zeros((NP, D), jnp.float32)
    p = _scatter_k(m, src, tgt, init)
    return _combine(x, p)


# speedup vs baseline: 1.7833x; 1.0237x over previous
import functools

import jax
import jax.numpy as jnp
from jax import lax
from jax.experimental import pallas as pl
from jax.experimental.pallas import tpu as pltpu
from jax.experimental.pallas import tpu_sc as plsc

N = 10000
E = 320000
D = 128

NC = 2
NS = 16
NW = NC * NS
E_PER_W = E // NW
B = 48
FULL_STEPS = 208
BT = E_PER_W - FULL_STEPS * B
NP = 10240
R_PER_T = NP // NS
RB = 80


def _msg_body(x_ref, wt_ref, m_ref):
    e = jnp.dot(x_ref[...], wt_ref[...], preferred_element_type=jnp.float32)
    g = jax.nn.sigmoid(e[:, :D])
    m_ref[...] = g * e[:, D:]


def _msg(x, wt):
    blk = 1000
    return pl.pallas_call(
        _msg_body,
        grid=(N // blk,),
        in_specs=[
            pl.BlockSpec((blk, D), lambda i: (i, 0)),
            pl.BlockSpec((D, 2 * D), lambda i: (0, 0)),
        ],
        out_specs=pl.BlockSpec((blk, D), lambda i: (i, 0)),
        out_shape=jax.ShapeDtypeStruct((N, D), jnp.float32),
    )(x, wt)


def _make_scatter():
    mesh = plsc.VectorSubcoreMesh(core_axis_name="c", subcore_axis_name="s")

    @functools.partial(
        pl.kernel,
        out_type=jax.ShapeDtypeStruct((NC, NP, D), jnp.float32),
        mesh=mesh,
        scratch_types=[
            pltpu.VMEM((E_PER_W,), jnp.int32),
            pltpu.VMEM((E_PER_W,), jnp.int32),
            pltpu.VMEM((B, D), jnp.float32),
            pltpu.VMEM((B, D), jnp.float32),
            pltpu.VMEM((B, D), jnp.float32),
            pltpu.VMEM((B, D), jnp.float32),
            pltpu.VMEM_SHARED((NP, D), jnp.float32),
            pltpu.SemaphoreType.DMA,
            pltpu.SemaphoreType.DMA,
            pltpu.SemaphoreType.DMA,
            pltpu.SemaphoreType.DMA,
        ],
    )
    def scatter_k(m_hbm, src_hbm, tgt_hbm, init_hbm, out_hbm,
                  tgt_v, src_v, rows0, rows1, rows2, rows3, acc,
                  semg0, semg1, semg2, semg3):
        c = lax.axis_index("c")
        s = lax.axis_index("s")
        wid = s * NC + c

        cp_t = pltpu.async_copy(tgt_hbm.at[wid], tgt_v, semg0)
        cp_s = pltpu.async_copy(src_hbm.at[wid], src_v, semg1)

        rbase = s * R_PER_T
        pltpu.sync_copy(init_hbm.at[pl.ds(rbase, R_PER_T)],
                        acc.at[pl.ds(rbase, R_PER_T)])
        cp_t.wait()
        cp_s.wait()
        plsc.subcore_barrier()

        def wait_g(buf, sem):
            pltpu.make_async_copy(m_hbm.at[pl.ds(0, B)], buf, sem).wait()

        def wait_s(buf, sem):
            pltpu.make_async_copy(buf, acc.at[pl.ds(0, B)], sem).wait()

        def tslice(ref, i):
            return ref.at[pl.ds(i * B, B)]

        bufs = [(rows0, semg0), (rows1, semg1), (rows2, semg2), (rows3, semg3)]
        ND = len(bufs)
        for j, (buf, sem) in enumerate(bufs):
            pltpu.async_copy(m_hbm.at[tslice(tgt_v, j)], buf, sem)

        def quad(t, carry):
            i = ND * t
            for j, (buf, sem) in enumerate(bufs):
                wait_g(buf, sem)
                pltpu.sync_copy(buf, acc.at[tslice(src_v, i + j)], add=True)

                @pl.when(i + j + ND < FULL_STEPS)
                def _():
                    pltpu.async_copy(
                        m_hbm.at[tslice(tgt_v, i + j + ND)], buf, sem)

            return carry

        lax.fori_loop(0, FULL_STEPS // ND, quad, 0)
        tb = FULL_STEPS * B
        pltpu.async_copy(
            m_hbm.at[tgt_v.at[pl.ds(tb, BT)]], rows0.at[pl.ds(0, BT)], semg0)
        pltpu.make_async_copy(
            m_hbm.at[pl.ds(0, BT)], rows0.at[pl.ds(0, BT)], semg0).wait()
        pltpu.sync_copy(
            rows0.at[pl.ds(0, BT)], acc.at[src_v.at[pl.ds(tb, BT)]], add=True)
        plsc.subcore_barrier()

        pltpu.sync_copy(acc.at[pl.ds(rbase, R_PER_T)],
                        out_hbm.at[c, pl.ds(rbase, R_PER_T)])

    return scatter_k


_scatter_k = _make_scatter()


def _add_body(x_ref, p_ref, o_ref):
    o_ref[...] = x_ref[...] + p_ref[0] + p_ref[1]


def _combine(x, p):
    blk = 1000
    return pl.pallas_call(
        _add_body,
        grid=(N // blk,),
        in_specs=[
            pl.BlockSpec((blk, D), lambda i: (i, 0)),
            pl.BlockSpec((NC, blk, D), lambda i: (0, i, 0)),
        ],
        out_specs=pl.BlockSpec((blk, D), lambda i: (i, 0)),
        out_shape=jax.ShapeDtypeStruct((N, D), jnp.float32),
    )(x, p)


def kernel(input, edge_sources, edge_targets, distance_nbr, W):
    x = input
    m = _msg(x, W.T)
    src = edge_sources.astype(jnp.int32).reshape(NW, E_PER_W)
    tgt = edge_targets.astype(jnp.int32).reshape(NW, E_PER_W)
    init = jnp.zeros((NP, D), jnp.float32)
    p = _scatter_k(m, src, tgt, init)
    return _combine(x, p)


# speedup vs baseline: 1.8099x; 1.0149x over previous
import functools

import jax
import jax.numpy as jnp
from jax import lax
from jax.experimental import pallas as pl
from jax.experimental.pallas import tpu as pltpu
from jax.experimental.pallas import tpu_sc as plsc

N = 10000
E = 320000
D = 128

NC = 2
NS = 16
NW = NC * NS
E_PER_W = E // NW
B = 40
FULL_STEPS = 250
NP = 10240
R_PER_T = NP // NS
RB = 80


def _msg_body(x_ref, wt_ref, m_ref):
    e = jnp.dot(x_ref[...], wt_ref[...], preferred_element_type=jnp.float32)
    g = jax.nn.sigmoid(e[:, :D])
    m_ref[...] = g * e[:, D:]


def _msg(x, wt):
    blk = 1000
    return pl.pallas_call(
        _msg_body,
        grid=(N // blk,),
        in_specs=[
            pl.BlockSpec((blk, D), lambda i: (i, 0)),
            pl.BlockSpec((D, 2 * D), lambda i: (0, 0)),
        ],
        out_specs=pl.BlockSpec((blk, D), lambda i: (i, 0)),
        out_shape=jax.ShapeDtypeStruct((N, D), jnp.float32),
    )(x, wt)


def _make_scatter():
    mesh = plsc.VectorSubcoreMesh(core_axis_name="c", subcore_axis_name="s")

    @functools.partial(
        pl.kernel,
        out_type=jax.ShapeDtypeStruct((NC, NP, D), jnp.float32),
        mesh=mesh,
        scratch_types=[
            pltpu.VMEM((E_PER_W,), jnp.int32),
            pltpu.VMEM((E_PER_W,), jnp.int32),
            pltpu.VMEM((B, D), jnp.float32),
            pltpu.VMEM((B, D), jnp.float32),
            pltpu.VMEM((B, D), jnp.float32),
            pltpu.VMEM((B, D), jnp.float32),
            pltpu.VMEM((B, D), jnp.float32),
            pltpu.VMEM_SHARED((NP, D), jnp.float32),
            pltpu.SemaphoreType.DMA,
            pltpu.SemaphoreType.DMA,
            pltpu.SemaphoreType.DMA,
            pltpu.SemaphoreType.DMA,
            pltpu.SemaphoreType.DMA,
        ],
    )
    def scatter_k(m_hbm, src_hbm, tgt_hbm, init_hbm, out_hbm,
                  tgt_v, src_v, rows0, rows1, rows2, rows3, rows4, acc,
                  semg0, semg1, semg2, semg3, semg4):
        c = lax.axis_index("c")
        s = lax.axis_index("s")
        wid = s * NC + c

        cp_t = pltpu.async_copy(tgt_hbm.at[wid], tgt_v, semg0)
        cp_s = pltpu.async_copy(src_hbm.at[wid], src_v, semg1)

        rbase = s * R_PER_T
        pltpu.sync_copy(init_hbm.at[pl.ds(rbase, R_PER_T)],
                        acc.at[pl.ds(rbase, R_PER_T)])
        cp_t.wait()
        cp_s.wait()
        plsc.subcore_barrier()

        def wait_g(buf, sem):
            pltpu.make_async_copy(m_hbm.at[pl.ds(0, B)], buf, sem).wait()

        def wait_s(buf, sem):
            pltpu.make_async_copy(buf, acc.at[pl.ds(0, B)], sem).wait()

        def tslice(ref, i):
            return ref.at[pl.ds(i * B, B)]

        bufs = [(rows0, semg0), (rows1, semg1), (rows2, semg2),
                (rows3, semg3), (rows4, semg4)]
        ND = len(bufs)
        for j, (buf, sem) in enumerate(bufs):
            pltpu.async_copy(m_hbm.at[tslice(tgt_v, j)], buf, sem)

        def quad(t, carry):
            i = ND * t
            for j, (buf, sem) in enumerate(bufs):
                wait_g(buf, sem)
                pltpu.sync_copy(buf, acc.at[tslice(src_v, i + j)], add=True)

                @pl.when(i + j + ND < FULL_STEPS)
                def _():
                    pltpu.async_copy(
                        m_hbm.at[tslice(tgt_v, i + j + ND)], buf, sem)

            return carry

        lax.fori_loop(0, FULL_STEPS // ND, quad, 0)
        plsc.subcore_barrier()

        pltpu.sync_copy(acc.at[pl.ds(rbase, R_PER_T)],
                        out_hbm.at[c, pl.ds(rbase, R_PER_T)])

    return scatter_k


_scatter_k = _make_scatter()


def _add_body(x_ref, p_ref, o_ref):
    o_ref[...] = x_ref[...] + p_ref[0] + p_ref[1]


def _combine(x, p):
    blk = 1000
    return pl.pallas_call(
        _add_body,
        grid=(N // blk,),
        in_specs=[
            pl.BlockSpec((blk, D), lambda i: (i, 0)),
            pl.BlockSpec((NC, blk, D), lambda i: (0, i, 0)),
        ],
        out_specs=pl.BlockSpec((blk, D), lambda i: (i, 0)),
        out_shape=jax.ShapeDtypeStruct((N, D), jnp.float32),
    )(x, p)


def kernel(input, edge_sources, edge_targets, distance_nbr, W):
    x = input
    m = _msg(x, W.T)
    src = edge_sources.astype(jnp.int32).reshape(NW, E_PER_W)
    tgt = edge_targets.astype(jnp.int32).reshape(NW, E_PER_W)
    init = jnp.zeros((NP, D), jnp.float32)
    p = _scatter_k(m, src, tgt, init)
    return _combine(x, p)
